# baseline (speedup 1.0000x reference)
import functools

import jax
import jax.numpy as jnp
from jax import lax
from jax.experimental import pallas as pl
from jax.experimental.pallas import tpu as pltpu
from jax.experimental.pallas import tpu_sc as plsc

NUM_CORES = 2
NUM_SUBCORES = 16
NUM_WORKERS = NUM_CORES * NUM_SUBCORES


@jax.jit
def _sc_gather(ids_t, table):
    seq, bsz = ids_t.shape
    d = table.shape[1]
    bpw = bsz // NUM_WORKERS
    mesh = plsc.VectorSubcoreMesh(core_axis_name="c", subcore_axis_name="s")

    half = bpw // 2
    nsteps = 2 * seq
    nbuf = 10
    lead = 5
    assert nsteps % nbuf == 0

    @functools.partial(
        pl.kernel,
        out_type=jax.ShapeDtypeStruct((seq, bsz, d), table.dtype),
        mesh=mesh,
        compiler_params=pltpu.CompilerParams(use_tc_tiling_on_sc=True),
        scratch_types=[
            pltpu.VMEM((seq, bpw), jnp.int32),
            pltpu.VMEM((nbuf, half, d), table.dtype),
            [pltpu.SemaphoreType.DMA] * nbuf,
            [pltpu.SemaphoreType.DMA] * nbuf,
        ],
    )
    def body(ids_hbm, table_hbm, out_hbm, idx_v, rows_v, gsems, wsems):
        wid = lax.axis_index("s") * NUM_CORES + lax.axis_index("c")
        base = wid * bpw
        pltpu.sync_copy(ids_hbm.at[:, pl.ds(base, bpw)], idx_v)

        def gather(s, h, b):
            return pltpu.make_async_copy(
                table_hbm.at[idx_v.at[s // 2, pl.ds(h * half, half)]],
                rows_v.at[b],
                gsems[b],
            )

        def writeback(s, h, b):
            return pltpu.make_async_copy(
                rows_v.at[b],
                out_hbm.at[s // 2, pl.ds(base + h * half, half)],
                wsems[b],
            )

        for s in range(lead):
            gather(s, s % 2, s).start()

        def outer(i, carry):
            for b in range(nbuf):
                s = nbuf * i + b
                bn = (b + lead) % nbuf

                @pl.when(s + lead < nsteps)
                def _():
                    @pl.when(s - (nbuf - lead) >= 0)
                    def _():
                        writeback(s - (nbuf - lead), bn % 2, bn).wait()

                    gather(s + lead, bn % 2, bn).start()

                gather(s, b % 2, b).wait()
                writeback(s, b % 2, b).start()
            return carry

        lax.fori_loop(0, nsteps // nbuf, outer, 0)
        for s in range(nsteps - nbuf, nsteps):
            writeback(s, s % 2, s % nbuf).wait()

    return body(ids_t, table)


def kernel(token_ids, weight):
    ids_t = token_ids.astype(jnp.int32).T
    out_t = _sc_gather(ids_t, weight)
    return jnp.transpose(out_t, (1, 0, 2))


# speedup vs baseline: 1.0037x; 1.0037x over previous
import functools

import jax
import jax.numpy as jnp
from jax import lax
from jax.experimental import pallas as pl
from jax.experimental.pallas import tpu as pltpu
from jax.experimental.pallas import tpu_sc as plsc

NUM_CORES = 2
NUM_SUBCORES = 16
NUM_WORKERS = NUM_CORES * NUM_SUBCORES


@jax.jit
def _sc_gather(ids_t, table):
    seq, bsz = ids_t.shape
    d = table.shape[1]
    bpw = bsz // NUM_WORKERS
    mesh = plsc.VectorSubcoreMesh(core_axis_name="c", subcore_axis_name="s")

    half = bpw // 2
    nsteps = 2 * seq
    nbuf = 10
    lead = 5
    assert nsteps % nbuf == 0

    @functools.partial(
        pl.kernel,
        out_type=jax.ShapeDtypeStruct((seq, bsz, d), table.dtype),
        mesh=mesh,
        compiler_params=pltpu.CompilerParams(use_tc_tiling_on_sc=True),
        scratch_types=[
            pltpu.VMEM((seq, bpw), jnp.int32),
            pltpu.VMEM((nbuf, half, d), table.dtype),
            [pltpu.SemaphoreType.DMA] * nbuf,
            [pltpu.SemaphoreType.DMA] * nbuf,
        ],
    )
    def body(ids_hbm, table_hbm, out_hbm, idx_v, rows_v, gsems, wsems):
        wid = lax.axis_index("s") * NUM_CORES + lax.axis_index("c")
        base = wid * bpw
        pltpu.sync_copy(ids_hbm.at[:, pl.ds(base, bpw)], idx_v)

        def gather(s, h, b):
            return pltpu.make_async_copy(
                table_hbm.at[idx_v.at[s // 2, pl.ds(h * half, half)]],
                rows_v.at[b],
                gsems[b],
            )

        def writeback(s, h, b):
            return pltpu.make_async_copy(
                rows_v.at[b],
                out_hbm.at[s // 2, pl.ds(base + h * half, half)],
                wsems[b],
            )

        for s in range(lead):
            gather(s, s % 2, s).start()

        def outer(i, carry):
            for b in range(nbuf):
                s = nbuf * i + b
                gather(s, b % 2, b).wait()
                writeback(s, b % 2, b).start()
                bn = (b + lead) % nbuf

                @pl.when(s + lead < nsteps)
                def _():
                    @pl.when(s - (nbuf - lead) >= 0)
                    def _():
                        writeback(s - (nbuf - lead), bn % 2, bn).wait()

                    gather(s + lead, bn % 2, bn).start()
            return carry

        lax.fori_loop(0, nsteps // nbuf, outer, 0)
        for s in range(nsteps - nbuf, nsteps):
            writeback(s, s % 2, s % nbuf).wait()

    return body(ids_t, table)


def kernel(token_ids, weight):
    ids_t = token_ids.astype(jnp.int32).T
    out_t = _sc_gather(ids_t, weight)
    return jnp.transpose(out_t, (1, 0, 2))
